# baseline (speedup 1.0000x reference)
import functools

import jax
import jax.numpy as jnp
from jax import lax
from jax.experimental import pallas as pl
from jax.experimental.pallas import tpu as pltpu
from jax.experimental.pallas import tpu_sc as plsc

LENGTH = 100000
D = 128
B = 16384
MOM = 0.5

NC, NS, LANES = 2, 16, 16
NW = NC * NS
BPW = B // NW
SHARD = 3136
LAST = LENGTH - (NW - 1) * SHARD
CHUNK = 128
LISTCAP = SHARD + CHUNK


def _mesh():
    return plsc.VectorSubcoreMesh(core_axis_name="c", subcore_axis_name="s")


def _iota16():
    return lax.iota(jnp.int32, 16)


def _sc_gather(bank, ind):
    @functools.partial(
        pl.kernel,
        out_type=jax.ShapeDtypeStruct((B, D), jnp.float32),
        mesh=_mesh(),
        compiler_params=pltpu.CompilerParams(needs_layout_passes=False),
        scratch_types=[
            pltpu.VMEM((BPW,), jnp.int32),
            pltpu.VMEM((BPW, D), jnp.float32),
            pltpu.SemaphoreType.DMA,
        ],
    )
    def k(bank_hbm, ind_hbm, out_hbm, idx_v, rows_v, sem):
        wid = lax.axis_index("s") * NC + lax.axis_index("c")
        base = wid * BPW
        pltpu.sync_copy(ind_hbm.at[pl.ds(base, BPW)], idx_v)
        pltpu.async_copy(bank_hbm.at[idx_v], rows_v, sem).wait()
        pltpu.sync_copy(rows_v, out_hbm.at[pl.ds(base, BPW)])

    return k(bank, ind)


def _tc_blend(old, feature):
    RB = 2048

    def body(o_ref, f_ref, out_ref):
        f = f_ref[...]
        o = o_ref[...]
        fn = f / (jnp.sqrt(jnp.sum(f * f, axis=1, keepdims=True)) + 1e-10)
        nw = MOM * o + (1.0 - MOM) * fn
        out_ref[...] = nw / (jnp.sqrt(jnp.sum(nw * nw, axis=1, keepdims=True)) + 1e-10)

    return pl.pallas_call(
        body,
        grid=(B // RB,),
        in_specs=[
            pl.BlockSpec((RB, D), lambda i: (i, 0)),
            pl.BlockSpec((RB, D), lambda i: (i, 0)),
        ],
        out_specs=pl.BlockSpec((RB, D), lambda i: (i, 0)),
        out_shape=jax.ShapeDtypeStruct((B, D), jnp.float32),
    )(old, feature)


def _sc_update(bank, labels, ind, fnew, newlab):
    @functools.partial(
        pl.kernel,
        out_type=(
            jax.ShapeDtypeStruct((LENGTH, D), jnp.float32),
            jax.ShapeDtypeStruct((LENGTH,), jnp.int32),
        ),
        mesh=_mesh(),
        compiler_params=pltpu.CompilerParams(needs_layout_passes=False),
        scratch_types=[
            pltpu.VMEM((B,), jnp.int32),
            pltpu.VMEM((B,), jnp.int32),
            pltpu.VMEM((SHARD,), jnp.int32),
            pltpu.VMEM((LISTCAP,), jnp.int32),
            pltpu.VMEM((LISTCAP,), jnp.int32),
            pltpu.VMEM((SHARD,), jnp.int32),
            pltpu.VMEM((CHUNK,), jnp.int32),
            pltpu.VMEM((CHUNK,), jnp.int32),
            pltpu.VMEM((CHUNK, D), jnp.float32),
            pltpu.SemaphoreType.DMA,
            pltpu.SemaphoreType.DMA,
            pltpu.SemaphoreType.DMA,
        ],
    )
    def k(bank_hbm, lab_hbm, ind_hbm, fnew_hbm, nlab_hbm,
          out_fb, out_lb,
          ind_v, labf_v, win_v, poslist, lidxlist, labsh_v,
          posbuf, bidxbuf, rowbuf, csem, gsem, ssem):
        wid = lax.axis_index("s") * NC + lax.axis_index("c")
        base = wid * SHARD
        hi = jnp.minimum(base + SHARD, LENGTH)
        is_last = wid == NW - 1

        @pl.when(jnp.logical_not(is_last))
        def _():
            pltpu.make_async_copy(
                bank_hbm.at[pl.ds(base, SHARD)],
                out_fb.at[pl.ds(base, SHARD)], csem).start()

        @pl.when(is_last)
        def _():
            pltpu.make_async_copy(
                bank_hbm.at[pl.ds(base, LAST)],
                out_fb.at[pl.ds(base, LAST)], csem).start()

        pltpu.sync_copy(ind_hbm, ind_v)
        pltpu.sync_copy(nlab_hbm, labf_v)

        @pl.when(jnp.logical_not(is_last))
        def _():
            pltpu.sync_copy(lab_hbm.at[pl.ds(base, SHARD)], labsh_v)

        @pl.when(is_last)
        def _():
            pltpu.sync_copy(lab_hbm.at[pl.ds(base, LAST)],
                            labsh_v.at[pl.ds(0, LAST)])

        iota = _iota16()

        with jax.named_scope("ph_zero"):
            def zero_body(t, _):
                win_v[pl.ds(t * 16, 16)] = jnp.zeros((16,), jnp.int32)
                return 0
            lax.fori_loop(0, SHARD // 16, zero_body, 0)

        with jax.named_scope("ph_scan"):
            def scan_body(j, _):
                v = ind_v[pl.ds(j * 16, 16)]
                pos1 = iota + (j * 16 + 1)
                m = jnp.logical_and(v >= base, v < hi)
                lidx = jnp.where(m, v - base, 0)
                plsc.store_scatter(win_v, [lidx], pos1, mask=m)
                return 0
            lax.fori_loop(0, B // 16, scan_body, 0)

        with jax.named_scope("ph_compact"):
          def cmp_body(t, off):
            wv = win_v[pl.ds(t * 16, 16)]
            m = wv > 0
            mi = m.astype(jnp.int32)
            pos = wv - 1
            tgt = off + plsc.cumsum(mi) - mi
            tgt = jnp.where(m, tgt, 0)
            plsc.store_scatter(poslist, [tgt], pos, mask=m)
            lrow = iota + t * 16
            plsc.store_scatter(lidxlist, [tgt], lrow, mask=m)
            lv = plsc.load_gather(labf_v, [jnp.where(m, pos, 0)], mask=m)
            cur = labsh_v[pl.ds(t * 16, 16)]
            labsh_v[pl.ds(t * 16, 16)] = jnp.where(m, lv, cur)
            return off + jnp.sum(mi)
          nw_cnt = lax.fori_loop(0, SHARD // 16, cmp_body, jnp.int32(0))

        p0 = poslist[pl.ds(0, 16)][0]
        l0 = lidxlist[pl.ds(0, 16)][0]

        with jax.named_scope("ph_pad"):
            def pad_body(t, _):
                gi = iota + t * 16
                pv = poslist[pl.ds(t * 16, 16)]
                lv = lidxlist[pl.ds(t * 16, 16)]
                sel = gi >= nw_cnt
                poslist[pl.ds(t * 16, 16)] = jnp.where(sel, p0, pv)
                lidxlist[pl.ds(t * 16, 16)] = jnp.where(sel, l0, lv)
                return 0
            lax.fori_loop(0, LISTCAP // 16, pad_body, 0)

        @pl.when(jnp.logical_not(is_last))
        def _():
            pltpu.make_async_copy(
                bank_hbm.at[pl.ds(base, SHARD)],
                out_fb.at[pl.ds(base, SHARD)], csem).wait()

        @pl.when(is_last)
        def _():
            pltpu.make_async_copy(
                bank_hbm.at[pl.ds(base, LAST)],
                out_fb.at[pl.ds(base, LAST)], csem).wait()

        nchunks = (nw_cnt + CHUNK - 1) // CHUNK

        with jax.named_scope("ph_chunks"):
            def chunk_body(i, _):
                c = i * CHUNK
                for kk in range(CHUNK // 16):
                    posbuf[pl.ds(kk * 16, 16)] = poslist[pl.ds(c + kk * 16, 16)]
                    bidxbuf[pl.ds(kk * 16, 16)] = (
                        lidxlist[pl.ds(c + kk * 16, 16)] + base)
                pltpu.async_copy(fnew_hbm.at[posbuf], rowbuf, gsem).wait()
                pltpu.async_copy(rowbuf, out_fb.at[bidxbuf], ssem).wait()
                return 0
            lax.fori_loop(0, nchunks, chunk_body, 0)

        @pl.when(jnp.logical_not(is_last))
        def _():
            pltpu.sync_copy(labsh_v, out_lb.at[pl.ds(base, SHARD)])

        @pl.when(is_last)
        def _():
            pltpu.sync_copy(labsh_v.at[pl.ds(0, LAST)],
                            out_lb.at[pl.ds(base, LAST)])

    return k(bank, labels, ind, fnew, newlab)


def kernel(feature_bank, label_bank, ind, feature, label):
    ind = ind.astype(jnp.int32)
    label = label.astype(jnp.int32)
    old = _sc_gather(feature_bank, ind)
    fnew = _tc_blend(old, feature)
    return _sc_update(feature_bank, label_bank, ind, fnew, label)


# speedup vs baseline: 11.5633x; 11.5633x over previous
import functools

import jax
import jax.numpy as jnp
from jax import lax
from jax.experimental import pallas as pl
from jax.experimental.pallas import tpu as pltpu
from jax.experimental.pallas import tpu_sc as plsc

LENGTH = 100000
D = 128
B = 16384
MOM = 0.5

NC, NS, LANES = 2, 16, 16
NW = NC * NS
BPW = B // NW
SHARD = 3200
LAST = LENGTH - (NW - 1) * SHARD
CHUNK = 128
LISTCAP = SHARD + CHUNK
CC = 200
MAXSTEPS = SHARD // CC


def _mesh():
    return plsc.VectorSubcoreMesh(core_axis_name="c", subcore_axis_name="s")


def _iota16():
    return lax.iota(jnp.int32, 16)


def _sc_gather(bank, ind):
    @functools.partial(
        pl.kernel,
        out_type=jax.ShapeDtypeStruct((B, D), jnp.float32),
        mesh=_mesh(),
        compiler_params=pltpu.CompilerParams(needs_layout_passes=False),
        scratch_types=[
            pltpu.VMEM((BPW,), jnp.int32),
            pltpu.VMEM((BPW, D), jnp.float32),
            pltpu.SemaphoreType.DMA,
        ],
    )
    def k(bank_hbm, ind_hbm, out_hbm, idx_v, rows_v, sem):
        wid = lax.axis_index("s") * NC + lax.axis_index("c")
        base = wid * BPW
        pltpu.sync_copy(ind_hbm.at[pl.ds(base, BPW)], idx_v)
        pltpu.async_copy(bank_hbm.at[idx_v], rows_v, sem).wait()
        pltpu.sync_copy(rows_v, out_hbm.at[pl.ds(base, BPW)])

    return k(bank, ind)


def _tc_blend(old, feature):
    RB = 2048

    def body(o_ref, f_ref, out_ref):
        f = f_ref[...]
        o = o_ref[...]
        fn = f / (jnp.sqrt(jnp.sum(f * f, axis=1, keepdims=True)) + 1e-10)
        nw = MOM * o + (1.0 - MOM) * fn
        out_ref[...] = nw / (jnp.sqrt(jnp.sum(nw * nw, axis=1, keepdims=True)) + 1e-10)

    return pl.pallas_call(
        body,
        grid=(B // RB,),
        in_specs=[
            pl.BlockSpec((RB, D), lambda i: (i, 0)),
            pl.BlockSpec((RB, D), lambda i: (i, 0)),
        ],
        out_specs=pl.BlockSpec((RB, D), lambda i: (i, 0)),
        out_shape=jax.ShapeDtypeStruct((B, D), jnp.float32),
    )(old, feature)


def _sc_update(bank, labels, ind, fnew, newlab):
    @functools.partial(
        pl.kernel,
        out_type=(
            jax.ShapeDtypeStruct((LENGTH, D), jnp.float32),
            jax.ShapeDtypeStruct((LENGTH,), jnp.int32),
        ),
        mesh=_mesh(),
        compiler_params=pltpu.CompilerParams(needs_layout_passes=False),
        scratch_types=[
            pltpu.VMEM((B,), jnp.int32),
            pltpu.VMEM((B,), jnp.int32),
            pltpu.VMEM((SHARD,), jnp.int32),
            pltpu.VMEM((LISTCAP,), jnp.int32),
            pltpu.VMEM((LISTCAP,), jnp.int32),
            pltpu.VMEM((SHARD,), jnp.int32),
            pltpu.VMEM((CHUNK,), jnp.int32),
            pltpu.VMEM((CHUNK,), jnp.int32),
            pltpu.VMEM((CHUNK, D), jnp.float32),
            pltpu.VMEM((CC, D), jnp.float32),
            pltpu.VMEM((CC, D), jnp.float32),
            pltpu.SemaphoreType.DMA,
            pltpu.SemaphoreType.DMA,
            pltpu.SemaphoreType.DMA,
            pltpu.SemaphoreType.DMA,
            pltpu.SemaphoreType.DMA,
            pltpu.SemaphoreType.DMA,
        ],
    )
    def k(bank_hbm, lab_hbm, ind_hbm, fnew_hbm, nlab_hbm,
          out_fb, out_lb,
          ind_v, labf_v, win_v, poslist, lidxlist, labsh_v,
          posbuf, bidxbuf, rowbuf, cb0, cb1, rs0, rs1, ws0, ws1,
          gsem, ssem):
        wid = lax.axis_index("s") * NC + lax.axis_index("c")
        base = wid * SHARD
        hi = jnp.minimum(base + SHARD, LENGTH)
        is_last = wid == NW - 1
        steps = jnp.where(is_last, LAST // CC, MAXSTEPS)
        cbufs, rsems, wsems = (cb0, cb1), (rs0, rs1), (ws0, ws1)

        def rd_desc(i, b):
            return pltpu.make_async_copy(
                bank_hbm.at[pl.ds(base + i * CC, CC)], cbufs[b], rsems[b])

        def wr_desc(i, b):
            return pltpu.make_async_copy(
                cbufs[b], out_fb.at[pl.ds(base + i * CC, CC)], wsems[b])

        rd_desc(0, 0).start()

        @pl.when(jnp.int32(1) < steps)
        def _():
            rd_desc(1, 1).start()

        pltpu.sync_copy(ind_hbm, ind_v)
        pltpu.sync_copy(nlab_hbm, labf_v)

        @pl.when(jnp.logical_not(is_last))
        def _():
            pltpu.sync_copy(lab_hbm.at[pl.ds(base, SHARD)], labsh_v)

        @pl.when(is_last)
        def _():
            pltpu.sync_copy(lab_hbm.at[pl.ds(base, LAST)],
                            labsh_v.at[pl.ds(0, LAST)])

        iota = _iota16()

        with jax.named_scope("ph_zero"):
            def zero_body(t, _):
                win_v[pl.ds(t * 16, 16)] = jnp.zeros((16,), jnp.int32)
                return 0
            lax.fori_loop(0, SHARD // 16, zero_body, 0)

        with jax.named_scope("ph_scan"):
            def scan_body(j, _):
                v = ind_v[pl.ds(j * 16, 16)]
                pos1 = iota + (j * 16 + 1)
                m = jnp.logical_and(v >= base, v < hi)
                lidx = jnp.where(m, v - base, 0)
                plsc.store_scatter(win_v, [lidx], pos1, mask=m)
                return 0
            lax.fori_loop(0, B // 16, scan_body, 0)

        with jax.named_scope("ph_compact"):
          def cmp_body(t, off):
            wv = win_v[pl.ds(t * 16, 16)]
            m = wv > 0
            mi = m.astype(jnp.int32)
            pos = wv - 1
            tgt = off + plsc.cumsum(mi) - mi
            tgt = jnp.where(m, tgt, 0)
            plsc.store_scatter(poslist, [tgt], pos, mask=m)
            lrow = iota + t * 16
            plsc.store_scatter(lidxlist, [tgt], lrow, mask=m)
            lv = plsc.load_gather(labf_v, [jnp.where(m, pos, 0)], mask=m)
            cur = labsh_v[pl.ds(t * 16, 16)]
            labsh_v[pl.ds(t * 16, 16)] = jnp.where(m, lv, cur)
            return off + jnp.sum(mi)
          nw_cnt = lax.fori_loop(0, SHARD // 16, cmp_body, jnp.int32(0))

        p0 = poslist[pl.ds(0, 16)][0]
        l0 = lidxlist[pl.ds(0, 16)][0]

        with jax.named_scope("ph_pad"):
            def pad_body(t, _):
                gi = iota + t * 16
                pv = poslist[pl.ds(t * 16, 16)]
                lv = lidxlist[pl.ds(t * 16, 16)]
                sel = gi >= nw_cnt
                poslist[pl.ds(t * 16, 16)] = jnp.where(sel, p0, pv)
                lidxlist[pl.ds(t * 16, 16)] = jnp.where(sel, l0, lv)
                return 0
            lax.fori_loop(0, LISTCAP // 16, pad_body, 0)

        with jax.named_scope("ph_copy"):
            for i in range(MAXSTEPS):
                b = i & 1

                @pl.when(jnp.int32(i) < steps)
                def _(i=i, b=b):
                    if i >= 2:
                        wr_desc(i - 2, b).wait()
                    if i >= 2:
                        rd_desc(i, b).start()
                    rd_desc(i, b).wait()
                    wr_desc(i, b).start()

            wr_desc(0, 0).wait()
            wr_desc(1, 1).wait()

        nchunks = (nw_cnt + CHUNK - 1) // CHUNK

        with jax.named_scope("ph_chunks"):
            def chunk_body(i, _):
                c = i * CHUNK
                for kk in range(CHUNK // 16):
                    posbuf[pl.ds(kk * 16, 16)] = poslist[pl.ds(c + kk * 16, 16)]
                    bidxbuf[pl.ds(kk * 16, 16)] = (
                        lidxlist[pl.ds(c + kk * 16, 16)] + base)
                pltpu.async_copy(fnew_hbm.at[posbuf], rowbuf, gsem).wait()
                pltpu.async_copy(rowbuf, out_fb.at[bidxbuf], ssem).wait()
                return 0
            lax.fori_loop(0, nchunks, chunk_body, 0)

        @pl.when(jnp.logical_not(is_last))
        def _():
            pltpu.sync_copy(labsh_v, out_lb.at[pl.ds(base, SHARD)])

        @pl.when(is_last)
        def _():
            pltpu.sync_copy(labsh_v.at[pl.ds(0, LAST)],
                            out_lb.at[pl.ds(base, LAST)])

    return k(bank, labels, ind, fnew, newlab)


def kernel(feature_bank, label_bank, ind, feature, label):
    ind = ind.astype(jnp.int32)
    label = label.astype(jnp.int32)
    old = _sc_gather(feature_bank, ind)
    fnew = _tc_blend(old, feature)
    return _sc_update(feature_bank, label_bank, ind, fnew, label)
